# baseline (speedup 1.0000x reference)
import functools

import jax
import jax.numpy as jnp
from jax import lax
from jax.experimental import pallas as pl
from jax.experimental.pallas import tpu as pltpu
from jax.experimental.pallas import tpu_sc as plsc

_NB = 4


def _sc_deg(dstp, zeros1, NP, CPT):
    mesh = plsc.VectorSubcoreMesh(core_axis_name="c", subcore_axis_name="s")
    half = CPT // 2

    @functools.partial(
        pl.kernel,
        mesh=mesh,
        out_type=jax.ShapeDtypeStruct((2 * NP,), jnp.float32),
        scratch_types=[
            pltpu.VMEM((CPT, 128), jnp.int32),
            pltpu.VMEM((128,), jnp.float32),
            pltpu.VMEM((NP // 16,), jnp.float32),
            pltpu.VMEM_SHARED((NP,), jnp.float32),
        ],
        compiler_params=pltpu.CompilerParams(use_tc_tiling_on_sc=False),
    )
    def k(dstp_hbm, z_hbm, out_hbm, dst_v, ones_v, wb_v, acc_sh):
        c = lax.axis_index("c")
        s = lax.axis_index("s")

        @pl.when(s == 0)
        def _():
            pltpu.sync_copy(z_hbm, acc_sh)

        pltpu.sync_copy(dstp_hbm.at[s], dst_v)
        for i in range(8):
            ones_v[pl.ds(i * 16, 16)] = jnp.ones((16,), jnp.float32)
        plsc.subcore_barrier()
        lo = c * half

        def body(j, carry):
            pltpu.sync_copy(ones_v, acc_sh.at[dst_v.at[lo + j]], add=True)
            return carry

        lax.fori_loop(0, half, body, 0)
        plsc.subcore_barrier()
        pt = NP // 16
        pltpu.sync_copy(acc_sh.at[pl.ds(s * pt, pt)], wb_v)
        pltpu.sync_copy(wb_v, out_hbm.at[pl.ds(c * NP + s * pt, pt)])

    return k(dstp, zeros1).reshape(2, NP)


def _sc_scatter_feat(table, srcp2, dstp, zeros, D, NP, CPT):
    mesh = plsc.VectorSubcoreMesh(core_axis_name="c", subcore_axis_name="s")

    @functools.partial(
        pl.kernel,
        mesh=mesh,
        out_type=jax.ShapeDtypeStruct((2, NP, D), jnp.float32),
        scratch_types=[
            pltpu.VMEM((CPT, 128), jnp.int32),
            pltpu.VMEM((CPT, 128), jnp.int32),
            [pltpu.VMEM((128, D), jnp.float32) for _ in range(_NB)],
            pltpu.VMEM_SHARED((NP, D), jnp.float32),
            [pltpu.SemaphoreType.DMA for _ in range(_NB)],
        ],
        compiler_params=pltpu.CompilerParams(use_tc_tiling_on_sc=False),
    )
    def k(t_hbm, srcp_hbm, dstp_hbm, z_hbm, out_hbm, src_v, dst_v, rows, acc_sh, sems):
        c = lax.axis_index("c")
        s = lax.axis_index("s")
        wid = c * 16 + s

        @pl.when(s == 0)
        def _():
            pltpu.sync_copy(z_hbm, acc_sh)

        pltpu.sync_copy(srcp_hbm.at[wid], src_v)
        pltpu.sync_copy(dstp_hbm.at[s], dst_v)
        plsc.subcore_barrier()

        for b in range(_NB):
            pltpu.async_copy(t_hbm.at[src_v.at[b]], rows[b], sems[b])

        def outer(jo, carry):
            j0 = jo * _NB
            for b in range(_NB):
                j = j0 + b
                pltpu.make_async_copy(t_hbm.at[src_v.at[j]], rows[b], sems[b]).wait()
                pltpu.sync_copy(rows[b], acc_sh.at[dst_v.at[j]], add=True)
                pltpu.async_copy(t_hbm.at[src_v.at[j + _NB]], rows[b], sems[b])
            return carry

        lax.fori_loop(0, CPT // _NB - 1, outer, 0)
        for b in range(_NB):
            j = CPT - _NB + b
            pltpu.make_async_copy(t_hbm.at[src_v.at[j]], rows[b], sems[b]).wait()
            pltpu.sync_copy(rows[b], acc_sh.at[dst_v.at[j]], add=True)
        plsc.subcore_barrier()
        pt = NP // 16
        nchunk = -(-pt // 128)
        for t in range(nchunk):
            nrows = min(128, pt - t * 128)
            base = s * pt + t * 128
            pltpu.sync_copy(acc_sh.at[pl.ds(base, nrows)], rows[0].at[pl.ds(0, nrows)])
            pltpu.sync_copy(rows[0].at[pl.ds(0, nrows)], out_hbm.at[c, pl.ds(base, nrows)])

    return k(table, srcp2, dstp, zeros)


def _sc_l2(table, srcp, dstp, zeros, dinvp, b2p, D, NP, CPT):
    mesh = plsc.VectorSubcoreMesh(core_axis_name="c", subcore_axis_name="s")

    @functools.partial(
        pl.kernel,
        mesh=mesh,
        out_type=jax.ShapeDtypeStruct((NP, D), jnp.float32),
        scratch_types=[
            pltpu.VMEM((CPT, 128), jnp.int32),
            pltpu.VMEM((CPT, 128), jnp.int32),
            [pltpu.VMEM((128, D), jnp.float32) for _ in range(_NB)],
            pltpu.VMEM((NP // 32, ), jnp.float32),
            pltpu.VMEM((16,), jnp.float32),
            pltpu.VMEM_SHARED((NP, D), jnp.float32),
            [pltpu.SemaphoreType.DMA for _ in range(_NB)],
        ],
        compiler_params=pltpu.CompilerParams(
            use_tc_tiling_on_sc=False, needs_layout_passes=False),
    )
    def k(t_hbm, srcp_hbm, dstp_hbm, z_hbm, dinv_hbm, b2_hbm, out_hbm,
          src_v, dst_v, rows, dinv_v, b2_v, acc_sh, sems):
        c = lax.axis_index("c")
        s = lax.axis_index("s")

        @pl.when(s == 0)
        def _():
            pltpu.sync_copy(z_hbm, acc_sh)

        pltpu.sync_copy(srcp_hbm.at[s], src_v)
        pltpu.sync_copy(dstp_hbm.at[s], dst_v)
        pt2 = NP // 32
        gbase = c * (NP // 2) + s * pt2
        pltpu.sync_copy(dinv_hbm.at[pl.ds(gbase, pt2)], dinv_v)
        pltpu.sync_copy(b2_hbm, b2_v)
        plsc.subcore_barrier()

        for b in range(_NB):
            pltpu.async_copy(t_hbm.at[src_v.at[b]], rows[b], sems[b])

        def outer(jo, carry):
            j0 = jo * _NB
            for b in range(_NB):
                j = j0 + b
                pltpu.make_async_copy(t_hbm.at[src_v.at[j]], rows[b], sems[b]).wait()
                pltpu.sync_copy(rows[b], acc_sh.at[dst_v.at[j]], add=True)
                pltpu.async_copy(t_hbm.at[src_v.at[j + _NB]], rows[b], sems[b])
            return carry

        lax.fori_loop(0, CPT // _NB - 1, outer, 0)
        for b in range(_NB):
            j = CPT - _NB + b
            pltpu.make_async_copy(t_hbm.at[src_v.at[j]], rows[b], sems[b]).wait()
            pltpu.sync_copy(rows[b], acc_sh.at[dst_v.at[j]], add=True)
        plsc.subcore_barrier()

        nchunk = -(-pt2 // 128)
        for t in range(nchunk):
            nrows = min(128, pt2 - t * 128)
            base = gbase + t * 128
            pltpu.sync_copy(acc_sh.at[pl.ds(base, nrows)], rows[0].at[pl.ds(0, nrows)])

            def erow(r, carry):
                idx = jnp.full((16,), t * 128 + r, jnp.int32)
                dv = plsc.load_gather(dinv_v, [idx])
                rows[0][r, :] = rows[0][r, :] * dv + b2_v[...]
                return carry

            lax.fori_loop(0, nrows, erow, 0)
            pltpu.sync_copy(rows[0].at[pl.ds(0, nrows)], out_hbm.at[pl.ds(base, nrows)])

    return k(table, srcp, dstp, zeros, dinvp, b2p)


def _p2_body(x_ref, w_ref, d0_ref, d1_ref, y_ref, dinv_ref):
    deg = d0_ref[...] + d1_ref[...]
    dinv = lax.rsqrt(jnp.maximum(deg, 1.0))
    xw = jnp.dot(x_ref[...], w_ref[0], preferred_element_type=jnp.float32)
    y_ref[...] = xw * dinv
    dinv_ref[...] = dinv


def _p4_body(p_ref, dinv_ref, b1_ref, w2_ref, y2_ref):
    z = jnp.concatenate([p_ref[0], p_ref[1]], axis=1) * dinv_ref[...] + b1_ref[...]
    h = jnp.maximum(z, 0.0)
    y2_ref[...] = jnp.dot(h, w2_ref[...], preferred_element_type=jnp.float32) * dinv_ref[...]


def kernel(x, edge_index, W1, b1, W2, b2):
    f32 = jnp.float32
    N, F = x.shape
    H = W1.shape[1]
    HH = H // 2
    C = W2.shape[1]
    E = edge_index.shape[1]
    E2 = E + N
    EPT = -(-E2 // 16)
    CPT = -(-EPT // 128)
    CPT = -(-CPT // (2 * _NB)) * (2 * _NB)
    NP = -(-(N + 1) // 256) * 256
    C16 = 16
    BN = 512
    G = -(-N // BN)
    NG = G * BN

    loop = jnp.arange(N, dtype=jnp.int32)
    pad = 16 * EPT - E2
    src = jnp.concatenate(
        [edge_index[0].astype(jnp.int32), loop, jnp.zeros((pad,), jnp.int32)]
    ).reshape(16, EPT)
    dst = jnp.concatenate(
        [edge_index[1].astype(jnp.int32), loop, jnp.full((pad,), N, jnp.int32)]
    ).reshape(16, EPT)
    pad2 = CPT * 128 - EPT
    srcp = jnp.pad(src, ((0, 0), (0, pad2))).reshape(16, CPT, 128)
    dstp = jnp.pad(dst, ((0, 0), (0, pad2)), constant_values=N).reshape(16, CPT, 128)
    srcp2 = jnp.concatenate([srcp, srcp + NG], axis=0)

    degp = _sc_deg(dstp, jnp.zeros((NP,), f32), NP, CPT)
    d0 = degp[0, :N, None]
    d1 = degp[1, :N, None]

    y_flat, dinv = pl.pallas_call(
        _p2_body,
        grid=(2, G),
        in_specs=[
            pl.BlockSpec((BN, F), lambda h, i: (i, 0)),
            pl.BlockSpec((1, F, HH), lambda h, i: (h, 0, 0)),
            pl.BlockSpec((BN, 1), lambda h, i: (i, 0)),
            pl.BlockSpec((BN, 1), lambda h, i: (i, 0)),
        ],
        out_specs=[
            pl.BlockSpec((BN, HH), lambda h, i: (h * G + i, 0)),
            pl.BlockSpec((BN, 1), lambda h, i: (i, 0)),
        ],
        out_shape=[
            jax.ShapeDtypeStruct((2 * NG, HH), f32),
            jax.ShapeDtypeStruct((N, 1), f32),
        ],
    )(x, jnp.stack([W1[:, :HH], W1[:, HH:]]), d0, d1)

    parts1 = _sc_scatter_feat(y_flat, srcp2, dstp, jnp.zeros((NP, HH), f32), HH, NP, CPT)

    W2p = jnp.zeros((H, C16), f32).at[:, :C].set(W2)
    y2 = pl.pallas_call(
        _p4_body,
        grid=(G,),
        in_specs=[
            pl.BlockSpec((2, BN, HH), lambda i: (0, i, 0)),
            pl.BlockSpec((BN, 1), lambda i: (i, 0)),
            pl.BlockSpec((1, H), lambda i: (0, 0)),
            pl.BlockSpec((H, C16), lambda i: (0, 0)),
        ],
        out_specs=pl.BlockSpec((BN, C16), lambda i: (i, 0)),
        out_shape=jax.ShapeDtypeStruct((N, C16), f32),
    )(parts1, dinv, b1.reshape(1, H), W2p)

    dinvp = jnp.pad(dinv[:, 0], (0, NP - N))
    b2p = jnp.zeros((C16,), f32).at[:C].set(b2)
    out_sc = _sc_l2(y2, srcp, dstp, jnp.zeros((NP, C16), f32), dinvp, b2p,
                    C16, NP, CPT)

    return out_sc[:N, :C]


# speedup vs baseline: 1.4925x; 1.4925x over previous
import functools

import jax
import jax.numpy as jnp
from jax import lax
from jax.experimental import pallas as pl
from jax.experimental.pallas import tpu as pltpu
from jax.experimental.pallas import tpu_sc as plsc

_NB = 4


def _sc_deg(dstp, zeros1, NP, CPT):
    mesh = plsc.VectorSubcoreMesh(core_axis_name="c", subcore_axis_name="s")
    half = CPT // 2

    @functools.partial(
        pl.kernel,
        mesh=mesh,
        out_type=jax.ShapeDtypeStruct((2 * NP,), jnp.float32),
        scratch_types=[
            pltpu.VMEM((CPT, 128), jnp.int32),
            pltpu.VMEM((128,), jnp.float32),
            pltpu.VMEM((NP // 16,), jnp.float32),
            pltpu.VMEM_SHARED((NP,), jnp.float32),
        ],
        compiler_params=pltpu.CompilerParams(use_tc_tiling_on_sc=False),
    )
    def k(dstp_hbm, z_hbm, out_hbm, dst_v, ones_v, wb_v, acc_sh):
        c = lax.axis_index("c")
        s = lax.axis_index("s")

        @pl.when(s == 0)
        def _():
            pltpu.sync_copy(z_hbm, acc_sh)

        pltpu.sync_copy(dstp_hbm.at[s], dst_v)
        for i in range(8):
            ones_v[pl.ds(i * 16, 16)] = jnp.ones((16,), jnp.float32)
        plsc.subcore_barrier()
        lo = c * half

        def body(j, carry):
            pltpu.sync_copy(ones_v, acc_sh.at[dst_v.at[lo + j]], add=True)
            return carry

        lax.fori_loop(0, half, body, 0)
        plsc.subcore_barrier()
        pt = NP // 16
        pltpu.sync_copy(acc_sh.at[pl.ds(s * pt, pt)], wb_v)
        pltpu.sync_copy(wb_v, out_hbm.at[pl.ds(c * NP + s * pt, pt)])

    return k(dstp, zeros1).reshape(2, NP)


def _sc_scatter_feat(table, srcp2, dstp, zeros, D, NP, CPT):
    mesh = plsc.VectorSubcoreMesh(core_axis_name="c", subcore_axis_name="s")

    @functools.partial(
        pl.kernel,
        mesh=mesh,
        out_type=jax.ShapeDtypeStruct((2, NP, D), jnp.float32),
        scratch_types=[
            pltpu.VMEM((CPT, 128), jnp.int32),
            pltpu.VMEM((CPT, 128), jnp.int32),
            [pltpu.VMEM((128, D), jnp.float32) for _ in range(_NB)],
            pltpu.VMEM_SHARED((NP, D), jnp.float32),
            [pltpu.SemaphoreType.DMA for _ in range(_NB)],
        ],
        compiler_params=pltpu.CompilerParams(use_tc_tiling_on_sc=False),
    )
    def k(t_hbm, srcp_hbm, dstp_hbm, z_hbm, out_hbm, src_v, dst_v, rows, acc_sh, sems):
        c = lax.axis_index("c")
        s = lax.axis_index("s")
        wid = c * 16 + s

        @pl.when(s == 0)
        def _():
            pltpu.sync_copy(z_hbm, acc_sh)

        pltpu.sync_copy(srcp_hbm.at[wid], src_v)
        pltpu.sync_copy(dstp_hbm.at[s], dst_v)
        plsc.subcore_barrier()

        for b in range(_NB):
            pltpu.async_copy(t_hbm.at[src_v.at[b]], rows[b], sems[b])

        def outer(jo, carry):
            j0 = jo * _NB
            for b in range(_NB):
                j = j0 + b
                pltpu.make_async_copy(t_hbm.at[src_v.at[j]], rows[b], sems[b]).wait()
                pltpu.sync_copy(rows[b], acc_sh.at[dst_v.at[j]], add=True)
                pltpu.async_copy(t_hbm.at[src_v.at[j + _NB]], rows[b], sems[b])
            return carry

        lax.fori_loop(0, CPT // _NB - 1, outer, 0)
        for b in range(_NB):
            j = CPT - _NB + b
            pltpu.make_async_copy(t_hbm.at[src_v.at[j]], rows[b], sems[b]).wait()
            pltpu.sync_copy(rows[b], acc_sh.at[dst_v.at[j]], add=True)
        plsc.subcore_barrier()
        pt = NP // 16
        nchunk = -(-pt // 128)
        for t in range(nchunk):
            nrows = min(128, pt - t * 128)
            base = s * pt + t * 128
            pltpu.sync_copy(acc_sh.at[pl.ds(base, nrows)], rows[0].at[pl.ds(0, nrows)])
            pltpu.sync_copy(rows[0].at[pl.ds(0, nrows)], out_hbm.at[c, pl.ds(base, nrows)])

    return k(table, srcp2, dstp, zeros)


def _sc_scatter_edge(table, srcp, dstp, zeros, D, NP, CPT):
    mesh = plsc.VectorSubcoreMesh(core_axis_name="c", subcore_axis_name="s")
    half = CPT // 2

    @functools.partial(
        pl.kernel,
        mesh=mesh,
        out_type=jax.ShapeDtypeStruct((2, NP, D), jnp.float32),
        scratch_types=[
            pltpu.VMEM((CPT, 128), jnp.int32),
            pltpu.VMEM((CPT, 128), jnp.int32),
            [pltpu.VMEM((128, D), jnp.float32) for _ in range(_NB)],
            pltpu.VMEM_SHARED((NP, D), jnp.float32),
            [pltpu.SemaphoreType.DMA for _ in range(_NB)],
        ],
        compiler_params=pltpu.CompilerParams(use_tc_tiling_on_sc=False),
    )
    def k(t_hbm, srcp_hbm, dstp_hbm, z_hbm, out_hbm, src_v, dst_v, rows, acc_sh, sems):
        c = lax.axis_index("c")
        s = lax.axis_index("s")

        @pl.when(s == 0)
        def _():
            pltpu.sync_copy(z_hbm, acc_sh)

        pltpu.sync_copy(srcp_hbm.at[s], src_v)
        pltpu.sync_copy(dstp_hbm.at[s], dst_v)
        plsc.subcore_barrier()
        lo = c * half

        for b in range(_NB):
            pltpu.async_copy(t_hbm.at[src_v.at[lo + b]], rows[b], sems[b])

        def outer(jo, carry):
            j0 = jo * _NB
            for b in range(_NB):
                j = lo + j0 + b
                pltpu.make_async_copy(t_hbm.at[src_v.at[j]], rows[b], sems[b]).wait()
                pltpu.sync_copy(rows[b], acc_sh.at[dst_v.at[j]], add=True)
                pltpu.async_copy(t_hbm.at[src_v.at[j + _NB]], rows[b], sems[b])
            return carry

        lax.fori_loop(0, half // _NB - 1, outer, 0)
        for b in range(_NB):
            j = lo + half - _NB + b
            pltpu.make_async_copy(t_hbm.at[src_v.at[j]], rows[b], sems[b]).wait()
            pltpu.sync_copy(rows[b], acc_sh.at[dst_v.at[j]], add=True)
        plsc.subcore_barrier()
        pt = NP // 16
        nchunk = -(-pt // 128)
        for t in range(nchunk):
            nrows = min(128, pt - t * 128)
            base = s * pt + t * 128
            pltpu.sync_copy(acc_sh.at[pl.ds(base, nrows)], rows[0].at[pl.ds(0, nrows)])
            pltpu.sync_copy(rows[0].at[pl.ds(0, nrows)], out_hbm.at[c, pl.ds(base, nrows)])

    return k(table, srcp, dstp, zeros)


def _p2_body(x_ref, w_ref, d0_ref, d1_ref, y_ref, dinv_ref):
    deg = d0_ref[...] + d1_ref[...] + 1.0
    dinv = lax.rsqrt(deg)
    xw = jnp.dot(x_ref[...], w_ref[0], preferred_element_type=jnp.float32)
    y_ref[...] = xw * dinv
    dinv_ref[...] = dinv


def _p4_body(p_ref, y0_ref, y1_ref, dinv_ref, b1_ref, w2_ref, y2_ref):
    z = jnp.concatenate([p_ref[0] + y0_ref[...], p_ref[1] + y1_ref[...]], axis=1)
    z = z * dinv_ref[...] + b1_ref[...]
    h = jnp.maximum(z, 0.0)
    y2_ref[...] = jnp.dot(h, w2_ref[...], preferred_element_type=jnp.float32) * dinv_ref[...]


def _p6_body(q_ref, y2_ref, dinv_ref, b2_ref, o_ref):
    o_ref[...] = (q_ref[0] + q_ref[1] + y2_ref[...]) * dinv_ref[...] + b2_ref[...]


def kernel(x, edge_index, W1, b1, W2, b2):
    f32 = jnp.float32
    N, F = x.shape
    H = W1.shape[1]
    HH = H // 2
    C = W2.shape[1]
    E = edge_index.shape[1]
    EPT = -(-E // 16)
    CPT = -(-EPT // 128)
    CPT = -(-CPT // (2 * _NB)) * (2 * _NB)
    NP = -(-(N + 1) // 128) * 128
    C16 = 16
    BN = 512
    G = -(-N // BN)
    NG = G * BN

    pad = CPT * 128 - EPT
    src = edge_index[0].astype(jnp.int32).reshape(16, EPT)
    dst = edge_index[1].astype(jnp.int32).reshape(16, EPT)
    srcp = jnp.pad(src, ((0, 0), (0, pad))).reshape(16, CPT, 128)
    dstp = jnp.pad(dst, ((0, 0), (0, pad)), constant_values=N).reshape(16, CPT, 128)
    srcp2 = jnp.concatenate([srcp, srcp + NG], axis=0)

    degp = _sc_deg(dstp, jnp.zeros((NP,), f32), NP, CPT)
    d0 = degp[0, :N, None]
    d1 = degp[1, :N, None]

    y_flat, dinv = pl.pallas_call(
        _p2_body,
        grid=(2, G),
        in_specs=[
            pl.BlockSpec((BN, F), lambda h, i: (i, 0)),
            pl.BlockSpec((1, F, HH), lambda h, i: (h, 0, 0)),
            pl.BlockSpec((BN, 1), lambda h, i: (i, 0)),
            pl.BlockSpec((BN, 1), lambda h, i: (i, 0)),
        ],
        out_specs=[
            pl.BlockSpec((BN, HH), lambda h, i: (h * G + i, 0)),
            pl.BlockSpec((BN, 1), lambda h, i: (i, 0)),
        ],
        out_shape=[
            jax.ShapeDtypeStruct((2 * NG, HH), f32),
            jax.ShapeDtypeStruct((N, 1), f32),
        ],
    )(x, jnp.stack([W1[:, :HH], W1[:, HH:]]), d0, d1)

    parts1 = _sc_scatter_feat(y_flat, srcp2, dstp, jnp.zeros((NP, HH), f32), HH, NP, CPT)

    W2p = jnp.zeros((H, C16), f32).at[:, :C].set(W2)
    y2 = pl.pallas_call(
        _p4_body,
        grid=(G,),
        in_specs=[
            pl.BlockSpec((2, BN, HH), lambda i: (0, i, 0)),
            pl.BlockSpec((BN, HH), lambda i: (i, 0)),
            pl.BlockSpec((BN, HH), lambda i: (G + i, 0)),
            pl.BlockSpec((BN, 1), lambda i: (i, 0)),
            pl.BlockSpec((1, H), lambda i: (0, 0)),
            pl.BlockSpec((H, C16), lambda i: (0, 0)),
        ],
        out_specs=pl.BlockSpec((BN, C16), lambda i: (i, 0)),
        out_shape=jax.ShapeDtypeStruct((N, C16), f32),
    )(parts1, y_flat, y_flat, dinv, b1.reshape(1, H), W2p)

    parts2 = _sc_scatter_edge(y2, srcp, dstp, jnp.zeros((NP, C16), f32), C16, NP, CPT)

    b2p = jnp.zeros((1, C16), f32).at[0, :C].set(b2)
    out16 = pl.pallas_call(
        _p6_body,
        grid=(G,),
        in_specs=[
            pl.BlockSpec((2, BN, C16), lambda i: (0, i, 0)),
            pl.BlockSpec((BN, C16), lambda i: (i, 0)),
            pl.BlockSpec((BN, 1), lambda i: (i, 0)),
            pl.BlockSpec((1, C16), lambda i: (0, 0)),
        ],
        out_specs=pl.BlockSpec((BN, C16), lambda i: (i, 0)),
        out_shape=jax.ShapeDtypeStruct((N, C16), f32),
    )(parts2, y2, dinv, b2p)

    return out16[:, :C]


# speedup vs baseline: 1.5848x; 1.0618x over previous
import functools

import jax
import jax.numpy as jnp
from jax import lax
from jax.experimental import pallas as pl
from jax.experimental.pallas import tpu as pltpu
from jax.experimental.pallas import tpu_sc as plsc

_NB = 4


def _sc_deg(dstp, zeros1, NP, CPT):
    mesh = plsc.VectorSubcoreMesh(core_axis_name="c", subcore_axis_name="s")
    half = CPT // 2

    @functools.partial(
        pl.kernel,
        mesh=mesh,
        out_type=jax.ShapeDtypeStruct((2 * NP,), jnp.float32),
        scratch_types=[
            pltpu.VMEM((CPT, 128), jnp.int32),
            pltpu.VMEM((128,), jnp.float32),
            pltpu.VMEM((NP // 16,), jnp.float32),
            pltpu.VMEM_SHARED((NP,), jnp.float32),
        ],
        compiler_params=pltpu.CompilerParams(use_tc_tiling_on_sc=False),
    )
    def k(dstp_hbm, z_hbm, out_hbm, dst_v, ones_v, wb_v, acc_sh):
        c = lax.axis_index("c")
        s = lax.axis_index("s")

        @pl.when(s == 0)
        def _():
            pltpu.sync_copy(z_hbm, acc_sh)

        pltpu.sync_copy(dstp_hbm.at[s], dst_v)
        for i in range(8):
            ones_v[pl.ds(i * 16, 16)] = jnp.ones((16,), jnp.float32)
        plsc.subcore_barrier()
        lo = c * half

        def body(j, carry):
            pltpu.sync_copy(ones_v, acc_sh.at[dst_v.at[lo + j]], add=True)
            return carry

        lax.fori_loop(0, half, body, 0)
        plsc.subcore_barrier()
        pt = NP // 16
        pltpu.sync_copy(acc_sh.at[pl.ds(s * pt, pt)], wb_v)
        pltpu.sync_copy(wb_v, out_hbm.at[pl.ds(c * NP + s * pt, pt)])

    return k(dstp, zeros1).reshape(2, NP)


def _sc_scatter_feat(table, srcp2, dstp, zeros, D, NP, CPT):
    mesh = plsc.VectorSubcoreMesh(core_axis_name="c", subcore_axis_name="s")

    @functools.partial(
        pl.kernel,
        mesh=mesh,
        out_type=jax.ShapeDtypeStruct((2, NP, D), jnp.float32),
        scratch_types=[
            pltpu.VMEM((CPT, 128), jnp.int32),
            pltpu.VMEM((CPT, 128), jnp.int32),
            [pltpu.VMEM((128, D), jnp.float32) for _ in range(_NB)],
            pltpu.VMEM_SHARED((NP, D), jnp.float32),
            [pltpu.SemaphoreType.DMA for _ in range(_NB)],
        ],
        compiler_params=pltpu.CompilerParams(use_tc_tiling_on_sc=False),
    )
    def k(t_hbm, srcp_hbm, dstp_hbm, z_hbm, out_hbm, src_v, dst_v, rows, acc_sh, sems):
        c = lax.axis_index("c")
        s = lax.axis_index("s")
        wid = c * 16 + s

        @pl.when(s == 0)
        def _():
            pltpu.sync_copy(z_hbm, acc_sh)

        pltpu.sync_copy(srcp_hbm.at[wid], src_v)
        pltpu.sync_copy(dstp_hbm.at[s], dst_v)
        plsc.subcore_barrier()

        for b in range(_NB):
            pltpu.async_copy(t_hbm.at[src_v.at[b]], rows[b], sems[b])

        def outer(jo, carry):
            j0 = jo * _NB
            for b in range(_NB):
                j = j0 + b
                pltpu.make_async_copy(t_hbm.at[src_v.at[j]], rows[b], sems[b]).wait()
                pltpu.sync_copy(rows[b], acc_sh.at[dst_v.at[j]], add=True)
                pltpu.async_copy(t_hbm.at[src_v.at[j + _NB]], rows[b], sems[b])
            return carry

        lax.fori_loop(0, CPT // _NB - 1, outer, 0)
        for b in range(_NB):
            j = CPT - _NB + b
            pltpu.make_async_copy(t_hbm.at[src_v.at[j]], rows[b], sems[b]).wait()
            pltpu.sync_copy(rows[b], acc_sh.at[dst_v.at[j]], add=True)
        plsc.subcore_barrier()
        pt = NP // 16
        nchunk = -(-pt // 128)
        for t in range(nchunk):
            nrows = min(128, pt - t * 128)
            base = s * pt + t * 128
            pltpu.sync_copy(acc_sh.at[pl.ds(base, nrows)], rows[0].at[pl.ds(0, nrows)])
            pltpu.sync_copy(rows[0].at[pl.ds(0, nrows)], out_hbm.at[c, pl.ds(base, nrows)])

    return k(table, srcp2, dstp, zeros)


def _sc_scatter_edge(table, srcp, dstp, zeros, D, NP, CPT):
    mesh = plsc.VectorSubcoreMesh(core_axis_name="c", subcore_axis_name="s")
    half = CPT // 2
    NCH = NP // 128

    @functools.partial(
        pl.kernel,
        mesh=mesh,
        out_type=jax.ShapeDtypeStruct((2, NP, D), jnp.float32),
        scratch_types=[
            pltpu.VMEM((CPT, 128), jnp.int32),
            pltpu.VMEM((CPT, 128), jnp.int32),
            [pltpu.VMEM((128, D), jnp.float32) for _ in range(_NB)],
            pltpu.VMEM_SHARED((NP, D), jnp.float32),
            pltpu.VMEM_SHARED((NP, D), jnp.float32),
            [pltpu.SemaphoreType.DMA for _ in range(_NB)],
        ],
        compiler_params=pltpu.CompilerParams(use_tc_tiling_on_sc=False),
    )
    def k(t_hbm, srcp_hbm, dstp_hbm, z_hbm, out_hbm, src_v, dst_v, rows, y_sh,
          acc_sh, sems):
        c = lax.axis_index("c")
        s = lax.axis_index("s")

        pltpu.sync_copy(z_hbm, rows[1])
        pltpu.sync_copy(srcp_hbm.at[s], src_v)
        pltpu.sync_copy(dstp_hbm.at[s], dst_v)
        for t in range(-(-NCH // 16)):
            ch = t * 16 + s

            @pl.when(ch < NCH)
            def _():
                pltpu.sync_copy(t_hbm.at[pl.ds(ch * 128, 128)], rows[0])
                pltpu.sync_copy(rows[0], y_sh.at[pl.ds(ch * 128, 128)])
                pltpu.sync_copy(rows[1], acc_sh.at[pl.ds(ch * 128, 128)])

        plsc.subcore_barrier()
        lo = c * half

        for b in range(_NB):
            pltpu.async_copy(y_sh.at[src_v.at[lo + b]], rows[b], sems[b])

        def outer(jo, carry):
            j0 = jo * _NB
            for b in range(_NB):
                j = lo + j0 + b
                pltpu.make_async_copy(y_sh.at[src_v.at[j]], rows[b], sems[b]).wait()
                pltpu.sync_copy(rows[b], acc_sh.at[dst_v.at[j]], add=True)
                pltpu.async_copy(y_sh.at[src_v.at[j + _NB]], rows[b], sems[b])
            return carry

        lax.fori_loop(0, half // _NB - 1, outer, 0)
        for b in range(_NB):
            j = lo + half - _NB + b
            pltpu.make_async_copy(y_sh.at[src_v.at[j]], rows[b], sems[b]).wait()
            pltpu.sync_copy(rows[b], acc_sh.at[dst_v.at[j]], add=True)
        plsc.subcore_barrier()
        pt = NP // 16
        nchunk = -(-pt // 128)
        for t in range(nchunk):
            nrows = min(128, pt - t * 128)
            base = s * pt + t * 128
            pltpu.sync_copy(acc_sh.at[pl.ds(base, nrows)], rows[0].at[pl.ds(0, nrows)])
            pltpu.sync_copy(rows[0].at[pl.ds(0, nrows)], out_hbm.at[c, pl.ds(base, nrows)])

    return k(table, srcp, dstp, zeros)


def _p2_body(x_ref, w_ref, d0_ref, d1_ref, y_ref, dinv_ref):
    deg = d0_ref[...] + d1_ref[...] + 1.0
    dinv = lax.rsqrt(deg)
    xw = jnp.dot(x_ref[...], w_ref[0], preferred_element_type=jnp.float32)
    y_ref[...] = xw * dinv
    dinv_ref[...] = dinv


def _p4_body(p_ref, y0_ref, y1_ref, dinv_ref, b1_ref, w2_ref, y2_ref):
    z = jnp.concatenate([p_ref[0] + y0_ref[...], p_ref[1] + y1_ref[...]], axis=1)
    z = z * dinv_ref[...] + b1_ref[...]
    h = jnp.maximum(z, 0.0)
    y2_ref[...] = jnp.dot(h, w2_ref[...], preferred_element_type=jnp.float32) * dinv_ref[...]


def _p6_body(q_ref, y2_ref, dinv_ref, b2_ref, o_ref):
    o_ref[...] = (q_ref[0] + q_ref[1] + y2_ref[...]) * dinv_ref[...] + b2_ref[...]


def kernel(x, edge_index, W1, b1, W2, b2):
    f32 = jnp.float32
    N, F = x.shape
    H = W1.shape[1]
    HH = H // 2
    C = W2.shape[1]
    E = edge_index.shape[1]
    EPT = -(-E // 16)
    CPT = -(-EPT // 128)
    CPT = -(-CPT // (2 * _NB)) * (2 * _NB)
    NP = -(-(N + 1) // 128) * 128
    C16 = 16
    BN = 512
    G = -(-N // BN)
    NG = G * BN

    pad = CPT * 128 - EPT
    src = edge_index[0].astype(jnp.int32).reshape(16, EPT)
    dst = edge_index[1].astype(jnp.int32).reshape(16, EPT)
    srcp = jnp.pad(src, ((0, 0), (0, pad))).reshape(16, CPT, 128)
    dstp = jnp.pad(dst, ((0, 0), (0, pad)), constant_values=N).reshape(16, CPT, 128)
    srcp2 = jnp.concatenate([srcp, srcp + NG], axis=0)

    degp = _sc_deg(dstp, jnp.zeros((NP,), f32), NP, CPT)
    d0 = degp[0, :N, None]
    d1 = degp[1, :N, None]

    y_flat, dinv = pl.pallas_call(
        _p2_body,
        grid=(2, G),
        in_specs=[
            pl.BlockSpec((BN, F), lambda h, i: (i, 0)),
            pl.BlockSpec((1, F, HH), lambda h, i: (h, 0, 0)),
            pl.BlockSpec((BN, 1), lambda h, i: (i, 0)),
            pl.BlockSpec((BN, 1), lambda h, i: (i, 0)),
        ],
        out_specs=[
            pl.BlockSpec((BN, HH), lambda h, i: (h * G + i, 0)),
            pl.BlockSpec((BN, 1), lambda h, i: (i, 0)),
        ],
        out_shape=[
            jax.ShapeDtypeStruct((2 * NG, HH), f32),
            jax.ShapeDtypeStruct((N, 1), f32),
        ],
    )(x, jnp.stack([W1[:, :HH], W1[:, HH:]]), d0, d1)

    parts1 = _sc_scatter_feat(y_flat, srcp2, dstp, jnp.zeros((NP, HH), f32), HH, NP, CPT)

    W2p = jnp.zeros((H, C16), f32).at[:, :C].set(W2)
    y2 = pl.pallas_call(
        _p4_body,
        grid=(G,),
        in_specs=[
            pl.BlockSpec((2, BN, HH), lambda i: (0, i, 0)),
            pl.BlockSpec((BN, HH), lambda i: (i, 0)),
            pl.BlockSpec((BN, HH), lambda i: (G + i, 0)),
            pl.BlockSpec((BN, 1), lambda i: (i, 0)),
            pl.BlockSpec((1, H), lambda i: (0, 0)),
            pl.BlockSpec((H, C16), lambda i: (0, 0)),
        ],
        out_specs=pl.BlockSpec((BN, C16), lambda i: (i, 0)),
        out_shape=jax.ShapeDtypeStruct((N, C16), f32),
    )(parts1, y_flat, y_flat, dinv, b1.reshape(1, H), W2p)

    y2p = jnp.pad(y2, ((0, NP - N), (0, 0)))
    parts2 = _sc_scatter_edge(y2p, srcp, dstp, jnp.zeros((128, C16), f32), C16, NP, CPT)

    b2p = jnp.zeros((1, C16), f32).at[0, :C].set(b2)
    out16 = pl.pallas_call(
        _p6_body,
        grid=(G,),
        in_specs=[
            pl.BlockSpec((2, BN, C16), lambda i: (0, i, 0)),
            pl.BlockSpec((BN, C16), lambda i: (i, 0)),
            pl.BlockSpec((BN, 1), lambda i: (i, 0)),
            pl.BlockSpec((1, C16), lambda i: (0, 0)),
        ],
        out_specs=pl.BlockSpec((BN, C16), lambda i: (i, 0)),
        out_shape=jax.ShapeDtypeStruct((N, C16), f32),
    )(parts2, y2, dinv, b2p)

    return out16[:, :C]
